# baseline (speedup 1.0000x reference)
import functools

import jax
import jax.numpy as jnp
from jax import lax
from jax.experimental import pallas as pl
from jax.experimental.pallas import tpu as pltpu
from jax.experimental.pallas import tpu_sc as plsc

B = 16384
F = 26
D = 16
M0 = 256
M1 = 128
EPS = 1e-5

NC = 2
NS = 16
NW = NC * NS
L = 16

TOTAL_IDX = B * F
IDX_PER_W = TOTAL_IDX // NW
SUB = 128
NSUB = IDX_PER_W // SUB
GRP = 8
NGRP = NSUB // GRP
GIDX = GRP * SUB


def _sc_gather_body(x2d, wemb, wfc16, emb_out, fc_out,
                    idx_v, idxs_v, rows_v, fcraw_v, fcv_v, sem_e, sem_f):
  wid = lax.axis_index("s") * NC + lax.axis_index("c")
  base_sub = wid * NSUB
  base_idx = wid * IDX_PER_W
  pltpu.sync_copy(x2d.at[pl.ds(base_sub, NSUB)], idx_v)

  def shift_body(r, c):
    for k in range(SUB // L):
      v = idx_v[r, pl.ds(k * L, L)]
      idxs_v[r, pl.ds(k * L, L)] = lax.shift_right_logical(v, 4)
    return c

  lax.fori_loop(0, NSUB, shift_body, 0, unroll=False)

  lane = lax.iota(jnp.int32, L)

  def group(g, carry):
    descs = []
    for j in range(GRP):
      row = g * GRP + j
      descs.append(pltpu.async_copy(
          wemb.at[idx_v.at[row]], rows_v.at[pl.ds(j * SUB, SUB)], sem_e))
      descs.append(pltpu.async_copy(
          wfc16.at[idxs_v.at[row]], fcraw_v.at[pl.ds(j * SUB, SUB)], sem_f))
    for d in descs:
      d.wait()
    for j in range(GRP):
      row = g * GRP + j
      for k in range(SUB // L):
        v = idx_v[row, pl.ds(k * L, L)]
        col = lax.bitwise_and(v, 15)
        rix = j * SUB + k * L + lane
        val = plsc.load_gather(fcraw_v, [rix, col])
        fcv_v[pl.ds(j * SUB + k * L, L)] = val
    off = base_idx + g * GIDX
    pltpu.sync_copy(rows_v, emb_out.at[pl.ds(off, GIDX)])
    pltpu.sync_copy(fcv_v, fc_out.at[pl.ds(off, GIDX)])
    return carry

  lax.fori_loop(0, NGRP, group, 0, unroll=False)


@jax.jit
def _sc_gather(x2d, wemb, wfc16):
  mesh = plsc.VectorSubcoreMesh(core_axis_name="c", subcore_axis_name="s",
                                num_cores=NC, num_subcores=NS)
  return pl.kernel(
      _sc_gather_body,
      out_type=(
          jax.ShapeDtypeStruct((TOTAL_IDX, D), jnp.float32),
          jax.ShapeDtypeStruct((TOTAL_IDX,), jnp.float32),
      ),
      mesh=mesh,
      scratch_types=[
          pltpu.VMEM((NSUB, SUB), jnp.int32),
          pltpu.VMEM((NSUB, SUB), jnp.int32),
          pltpu.VMEM((GIDX, D), jnp.float32),
          pltpu.VMEM((GIDX, D), jnp.float32),
          pltpu.VMEM((GIDX,), jnp.float32),
          pltpu.SemaphoreType.DMA,
          pltpu.SemaphoreType.DMA,
      ],
      compiler_params=pltpu.CompilerParams(use_tc_tiling_on_sc=False,
                                           needs_layout_passes=False),
  )(x2d, wemb, wfc16)


BM = 1024


def _tc_body(conc_ref, fcm_ref, w0t_ref, b0_ref, w1t_ref, b1_ref, wl_ref,
             lw_ref, bt_ref, out_ref):
  c = conc_ref[...]
  h = jnp.dot(c, w0t_ref[...], preferred_element_type=jnp.float32)
  h = jnp.maximum(h + b0_ref[...], 0.0)
  h = jnp.dot(h, w1t_ref[...], preferred_element_type=jnp.float32)
  h = jnp.maximum(h + b1_ref[...], 0.0)
  mlp = jnp.sum(h * wl_ref[...], axis=1, keepdims=True)
  s = c[:, 0:D]
  ss = s * s
  for f in range(1, F):
    e = c[:, f * D:(f + 1) * D]
    s = s + e
    ss = ss + e * e
  fm = 0.5 * jnp.sum(s * s - ss, axis=1, keepdims=True)
  lin = jnp.sum(fcm_ref[...], axis=1, keepdims=True) * lw_ref[0, 0]
  out_ref[...] = jax.nn.sigmoid(lin + fm + mlp + bt_ref[0, 0])


def _tc_head(conc, fcm, w0t, b0f, w1t, b1f, wlr, lw, btot):
  grid = (B // BM,)
  return pl.pallas_call(
      _tc_body,
      grid=grid,
      in_specs=[
          pl.BlockSpec((BM, F * D), lambda i: (i, 0)),
          pl.BlockSpec((BM, F), lambda i: (i, 0)),
          pl.BlockSpec((F * D, M0), lambda i: (0, 0)),
          pl.BlockSpec((1, M0), lambda i: (0, 0)),
          pl.BlockSpec((M0, M1), lambda i: (0, 0)),
          pl.BlockSpec((1, M1), lambda i: (0, 0)),
          pl.BlockSpec((1, M1), lambda i: (0, 0)),
          pl.BlockSpec(memory_space=pltpu.SMEM),
          pl.BlockSpec(memory_space=pltpu.SMEM),
      ],
      out_specs=pl.BlockSpec((BM, 1), lambda i: (i, 0)),
      out_shape=jax.ShapeDtypeStruct((B, 1), jnp.float32),
  )(conc, fcm, w0t, b0f, w1t, b1f, wlr, lw, btot)


def kernel(x, W_emb, W_fc, lin_w, lin_b, W0, b0, g0, bt0, rm0, rv0,
           W1, b1, g1, bt1, rm1, rv1, Wl, bl):
  x2d = x.reshape(TOTAL_IDX // SUB, SUB).astype(jnp.int32)
  wfc16 = W_fc.reshape(-1, 16)
  emb_flat, fc_flat = _sc_gather(x2d, W_emb, wfc16)
  conc = emb_flat.reshape(B, F * D)
  fcm = fc_flat.reshape(B, F)
  s0 = g0 * lax.rsqrt(rv0 + EPS)
  w0t = (W0 * s0[:, None]).T
  b0f = ((b0 - rm0) * s0 + bt0).reshape(1, M0)
  s1 = g1 * lax.rsqrt(rv1 + EPS)
  w1t = (W1 * s1[:, None]).T
  b1f = ((b1 - rm1) * s1 + bt1).reshape(1, M1)
  wlr = Wl.reshape(1, M1)
  btot = (lin_b + bl).reshape(1, 1)
  out = _tc_head(conc, fcm, w0t, b0f, w1t, b1f, wlr, lin_w, btot)
  return out[:, 0]
